# Initial kernel scaffold; baseline (speedup 1.0000x reference)
import jax
import jax.numpy as jnp
from jax.experimental import pallas as pl


def kernel(x, pos, batch, params):
    raise NotImplementedError("write your pallas kernel here")


# speedup vs baseline: 1.0025x; 1.0025x over previous
import jax
import jax.numpy as jnp
from jax.experimental import pallas as pl

CUTOFF = 5.0
NUM_GAUSSIANS = 50
HIDDEN_DIM = 128
NUM_FILTERS = 128
NUM_INTER = 3
OUTPUT_DIM = 250
NUM_GRAPHS = 128
E_MAX = 262144


def _ssp(x):
    return jax.nn.softplus(x) - jnp.log(2.0)


def _head_kernel(h_ref, batch_ref, w0_ref, b0_ref, w1_ref, b1_ref,
                 w2_ref, b2_ref, out_ref):
    h = h_ref[...]
    batch = batch_ref[...]
    gids = jax.lax.broadcasted_iota(jnp.int32, (NUM_GRAPHS, batch.shape[1]), 0)
    onehot = (batch == gids).astype(jnp.float32)
    hg = jnp.dot(onehot, h, preferred_element_type=jnp.float32)
    o = _ssp(jnp.dot(hg, w0_ref[...], preferred_element_type=jnp.float32)
             + b0_ref[...])
    o = _ssp(jnp.dot(o, w1_ref[...], preferred_element_type=jnp.float32)
             + b1_ref[...])
    o = jax.nn.sigmoid(jnp.dot(o, w2_ref[...], preferred_element_type=jnp.float32)
                       + b2_ref[...])
    out_ref[...] = o


def kernel(x, pos, batch, params):
    N = x.shape[0]
    sq = jnp.sum(pos * pos, axis=1)
    dist2 = jnp.maximum(sq[:, None] + sq[None, :] - 2.0 * (pos @ pos.T), 0.0)
    same = batch[:, None] == batch[None, :]
    mask = same & (dist2 < CUTOFF ** 2) & (dist2 > 1e-9)
    row, col = jnp.nonzero(mask, size=E_MAX, fill_value=0)
    valid = mask[row, col].astype(jnp.float32)
    diff = pos[row] - pos[col]
    dist = jnp.sqrt(jnp.sum(diff * diff, axis=-1) + 1e-12)
    offset = jnp.linspace(0.0, CUTOFF, NUM_GAUSSIANS)
    coeff = -0.5 / (CUTOFF / (NUM_GAUSSIANS - 1)) ** 2
    edge_attr = jnp.exp(coeff * (dist[:, None] - offset[None, :]) ** 2)
    h = x @ params['emb'][0] + params['emb'][1]
    for l in range(NUM_INTER):
        p = params['inter'][l]
        W = edge_attr @ p['f1'][0] + p['f1'][1]
        W = _ssp(W)
        W = W @ p['f2'][0] + p['f2'][1]
        x_lin = h @ p['lin1'][0] + p['lin1'][1]
        msg = x_lin[col] * W * valid[:, None]
        agg = jnp.zeros((N, NUM_FILTERS), dtype=h.dtype).at[row].add(msg)
        hc = agg @ p['lin2'][0] + p['lin2'][1]
        hc = _ssp(hc)
        hc = hc @ p['out'][0] + p['out'][1]
        h = h + hc

    out = pl.pallas_call(
        _head_kernel,
        out_shape=jax.ShapeDtypeStruct((NUM_GRAPHS, OUTPUT_DIM), jnp.float32),
    )(h, batch.astype(jnp.int32).reshape(1, N),
      params['mlp0'][0], params['mlp0'][1].reshape(1, -1),
      params['mlp1'][0], params['mlp1'][1].reshape(1, -1),
      params['mlp2'][0], params['mlp2'][1].reshape(1, -1))
    return out


# speedup vs baseline: 3.2445x; 3.2366x over previous
import functools

import jax
import jax.numpy as jnp
from jax import lax
from jax.experimental import pallas as pl
from jax.experimental.pallas import tpu as pltpu
from jax.experimental.pallas import tpu_sc as plsc

CUTOFF = 5.0
NUM_GAUSSIANS = 50
HIDDEN_DIM = 128
NUM_FILTERS = 128
NUM_INTER = 3
OUTPUT_DIM = 250
NUM_GRAPHS = 128
N_ATOMS = 4096
E_MAX = 262144
E_BUF = 263168
BLK = 1024
NBLK = E_BUF // BLK
NW = 32
APW = N_ATOMS // NW
STG = 2080
GCOEF = -0.5 / (CUTOFF / (NUM_GAUSSIANS - 1)) ** 2
LN2 = 0.6931471805599453

_SC_PARAMS = pltpu.CompilerParams(needs_layout_passes=False)
_VMESH = lambda: plsc.VectorSubcoreMesh(core_axis_name="c", subcore_axis_name="s")
_SDS = jax.ShapeDtypeStruct


def _ssp(x):
    return jnp.maximum(x, 0.0) + jnp.log1p(jnp.exp(-jnp.abs(x))) - LN2


def _wid():
    return lax.axis_index("s") * 2 + lax.axis_index("c")


def _vscal(ref, i):
    return ref[pl.ds(i, 16)][0]


def _popcnt(m):
    pc = plsc.all_reduce_population_count(m)
    return pc[0] if getattr(pc, "ndim", 0) else pc


def _pair_groups(pxv, pyv, pzv, i_glob, s, e):
    xi = jnp.full((16,), _vscal(pxv, i_glob), jnp.float32)
    yi = jnp.full((16,), _vscal(pyv, i_glob), jnp.float32)
    zi = jnp.full((16,), _vscal(pzv, i_glob), jnp.float32)
    base16 = (s // 16) * 16
    ng = (e - base16 + 15) // 16
    def group(g):
        j0 = base16 + g * 16
        jj = j0 + lax.iota(jnp.int32, 16)
        dx = pxv[pl.ds(j0, 16)] - xi
        dy = pyv[pl.ds(j0, 16)] - yi
        dz = pzv[pl.ds(j0, 16)] - zi
        d2 = dx * dx + dy * dy + dz * dz
        m = (jj >= s) & (jj < e) & (d2 < CUTOFF * CUTOFF) & (d2 > 1e-9)
        return jj, d2, m
    return ng, group


def _sc_count_fn():
    @functools.partial(
        pl.kernel,
        out_type=_SDS((N_ATOMS,), jnp.int32),
        mesh=_VMESH(),
        compiler_params=_SC_PARAMS,
        scratch_types=[
            pltpu.VMEM((4112,), jnp.float32),
            pltpu.VMEM((4112,), jnp.float32),
            pltpu.VMEM((4112,), jnp.float32),
            pltpu.VMEM((144,), jnp.int32),
            pltpu.VMEM((144,), jnp.int32),
            pltpu.VMEM((144,), jnp.int32),
        ],
    )
    def k(px_hbm, py_hbm, pz_hbm, ss_hbm, se_hbm, cnt_hbm,
          pxv, pyv, pzv, ssv, sev, cbv):
        w = _wid()
        a0 = w * APW
        pltpu.sync_copy(px_hbm, pxv.at[pl.ds(0, N_ATOMS)])
        pltpu.sync_copy(py_hbm, pyv.at[pl.ds(0, N_ATOMS)])
        pltpu.sync_copy(pz_hbm, pzv.at[pl.ds(0, N_ATOMS)])
        pltpu.sync_copy(ss_hbm.at[pl.ds(a0, APW)], ssv.at[pl.ds(0, APW)])
        pltpu.sync_copy(se_hbm.at[pl.ds(a0, APW)], sev.at[pl.ds(0, APW)])

        def atom(i, carry):
            s = _vscal(ssv, i)
            e = _vscal(sev, i)
            ng, group = _pair_groups(pxv, pyv, pzv, a0 + i, s, e)
            def body(g, cnt):
                _, _, m = group(g)
                return cnt + _popcnt(m)
            cnt = lax.fori_loop(0, ng, body, jnp.int32(0))
            tgt = jnp.where(lax.iota(jnp.int32, 16) == 0, i, 143)
            plsc.store_scatter(cbv, [tgt], jnp.full((16,), cnt, jnp.int32))
            return carry
        lax.fori_loop(0, APW, atom, jnp.int32(0))
        pltpu.sync_copy(cbv.at[pl.ds(0, APW)], cnt_hbm.at[pl.ds(a0, APW)])
    return k


def _sc_build_fn():
    @functools.partial(
        pl.kernel,
        out_type=(_SDS((E_BUF,), jnp.int32), _SDS((E_BUF,), jnp.float32)),
        mesh=_VMESH(),
        compiler_params=_SC_PARAMS,
        scratch_types=[
            pltpu.VMEM((4112,), jnp.float32),
            pltpu.VMEM((4112,), jnp.float32),
            pltpu.VMEM((4112,), jnp.float32),
            pltpu.VMEM((144,), jnp.int32),
            pltpu.VMEM((144,), jnp.int32),
            pltpu.VMEM((144,), jnp.int32),
            pltpu.VMEM((144,), jnp.int32),
            pltpu.VMEM((STG,), jnp.int32),
            pltpu.VMEM((STG,), jnp.float32),
        ],
    )
    def k(px_hbm, py_hbm, pz_hbm, ss_hbm, se_hbm, off_hbm, cl_hbm,
          col_hbm, d2_hbm,
          pxv, pyv, pzv, ssv, sev, offv, clv, scol, sd2):
        w = _wid()
        a0 = w * APW
        pltpu.sync_copy(px_hbm, pxv.at[pl.ds(0, N_ATOMS)])
        pltpu.sync_copy(py_hbm, pyv.at[pl.ds(0, N_ATOMS)])
        pltpu.sync_copy(pz_hbm, pzv.at[pl.ds(0, N_ATOMS)])
        pltpu.sync_copy(ss_hbm.at[pl.ds(a0, APW)], ssv.at[pl.ds(0, APW)])
        pltpu.sync_copy(se_hbm.at[pl.ds(a0, APW)], sev.at[pl.ds(0, APW)])
        pltpu.sync_copy(off_hbm.at[pl.ds(a0, APW)], offv.at[pl.ds(0, APW)])
        pltpu.sync_copy(cl_hbm.at[pl.ds(a0, APW)], clv.at[pl.ds(0, APW)])
        gbase = pl.multiple_of(_vscal(offv, 0), 8)

        def flush_if(spos, fl):
            def do(c):
                spos, fl = c
                ptr = pl.multiple_of(gbase + fl * 2048, 8)
                pltpu.sync_copy(scol.at[pl.ds(0, 2048)],
                                col_hbm.at[pl.ds(ptr, 2048)])
                pltpu.sync_copy(sd2.at[pl.ds(0, 2048)],
                                d2_hbm.at[pl.ds(ptr, 2048)])
                v0 = scol[pl.ds(2048, 16)]
                v1 = scol[pl.ds(2064, 16)]
                scol[pl.ds(0, 16)] = v0
                scol[pl.ds(16, 16)] = v1
                u0 = sd2[pl.ds(2048, 16)]
                u1 = sd2[pl.ds(2064, 16)]
                sd2[pl.ds(0, 16)] = u0
                sd2[pl.ds(16, 16)] = u1
                return (spos - 2048, fl + 1)
            return lax.cond(spos >= 2048, do, lambda c: c, (spos, fl))

        def atom(i, fl):
            s = _vscal(ssv, i)
            e = _vscal(sev, i)
            off = _vscal(offv, i)
            cl = _vscal(clv, i)
            spos0 = off - gbase - fl * 2048
            ng, group = _pair_groups(pxv, pyv, pzv, a0 + i, s, e)

            def grp(g, c):
                spos, app, fl = c
                jj, d2, m = group(g)
                mi = jnp.where(m, jnp.int32(1), jnp.int32(0))
                cs = plsc.cumsum(mi)
                m2 = m & ((cs + app) <= cl)
                mi2 = jnp.where(m2, jnp.int32(1), jnp.int32(0))
                cs2 = plsc.cumsum(mi2)
                tgt = jnp.where(m2, spos + cs2 - 1, jnp.int32(STG - 1))
                plsc.store_scatter(scol, [tgt], jj)
                plsc.store_scatter(sd2, [tgt], d2)
                pc = _popcnt(m2)
                spos, fl = flush_if(spos + pc, fl)
                return (spos, app + pc, fl)

            spos, app, fl = lax.fori_loop(0, ng, grp, (spos0, jnp.int32(0), fl))
            scol[pl.ds(spos, 16)] = jnp.zeros((16,), jnp.int32)
            sd2[pl.ds(spos, 16)] = jnp.zeros((16,), jnp.float32)
            spos, fl = flush_if(spos, fl)
            return fl

        fl = lax.fori_loop(0, APW, atom, jnp.int32(0))

        off_l = _vscal(offv, APW - 1)
        cl_l = _vscal(clv, APW - 1)
        end = off_l + ((cl_l + 7) // 8) * 8
        rem = end - gbase - fl * 2048
        ptr = gbase + fl * 2048
        n64 = rem // 64
        def f64(i, _):
            p = pl.multiple_of(ptr + i * 64, 8)
            pltpu.sync_copy(scol.at[pl.ds(i * 64, 64)], col_hbm.at[pl.ds(p, 64)])
            pltpu.sync_copy(sd2.at[pl.ds(i * 64, 64)], d2_hbm.at[pl.ds(p, 64)])
            return _
        lax.fori_loop(0, n64, f64, jnp.int32(0))
        t0 = n64 * 64
        n8 = (rem - t0) // 8
        def f8(i, _):
            p = pl.multiple_of(ptr + t0 + i * 8, 8)
            pltpu.sync_copy(scol.at[pl.ds(t0 + i * 8, 8)], col_hbm.at[pl.ds(p, 8)])
            pltpu.sync_copy(sd2.at[pl.ds(t0 + i * 8, 8)], d2_hbm.at[pl.ds(p, 8)])
            return _
        lax.fori_loop(0, n8, f8, jnp.int32(0))

        @pl.when(w == NW - 1)
        def _():
            scol[pl.ds(0, 16)] = jnp.zeros((16,), jnp.int32)
            scol[pl.ds(16, 16)] = jnp.zeros((16,), jnp.int32)
            scol[pl.ds(32, 16)] = jnp.zeros((16,), jnp.int32)
            scol[pl.ds(48, 16)] = jnp.zeros((16,), jnp.int32)
            p = pl.multiple_of(end, 8)
            pltpu.sync_copy(scol.at[pl.ds(0, 64)], col_hbm.at[pl.ds(p, 64)])
    return k


def _emb_body(x_ref, ew_ref, eb_ref, l1w_ref, l1b_ref, c_ref,
              h_ref, xl_ref, off_ref, cl_ref, nb_ref):
    h = jnp.dot(x_ref[...], ew_ref[...], preferred_element_type=jnp.float32)
    h = h + eb_ref[...]
    h_ref[...] = h
    xl_ref[...] = jnp.dot(h, l1w_ref[...],
                          preferred_element_type=jnp.float32) + l1b_ref[...]

    c = c_ref[...]
    c8 = jnp.bitwise_and(c + 7, -8)
    x = c8
    for sft in (1, 2, 4, 8, 16, 32, 64):
        sh = jnp.concatenate(
            [jnp.zeros((32, sft), jnp.int32), x[:, :128 - sft]], axis=1)
        x = x + sh
    ci = x
    row_tot = ci[:, 127:128]
    y = row_tot
    for sft in (1, 2, 4, 8, 16):
        sh = jnp.concatenate(
            [jnp.zeros((sft, 1), jnp.int32), y[:32 - sft, :]], axis=0)
        y = y + sh
    rb = y - row_tot
    off_excl = ci + rb - c8
    offc = jnp.minimum(off_excl, E_MAX)
    clc = jnp.minimum(c, jnp.maximum(E_MAX - offc, 0))
    off_ref[...] = offc
    cl_ref[...] = clc
    total8 = jnp.minimum(rb[31, 0] + row_tot[31, 0], E_MAX + 8)
    nb_ref[0, 0] = (total8 + BLK - 1) // BLK


def _emb_call(x, ew, eb, l1w, l1b, counts):
    return pl.pallas_call(
        _emb_body,
        out_shape=(
            _SDS((N_ATOMS, HIDDEN_DIM), jnp.float32),
            _SDS((N_ATOMS, NUM_FILTERS), jnp.float32),
            _SDS((32, 128), jnp.int32),
            _SDS((32, 128), jnp.int32),
            _SDS((1, 1), jnp.int32),
        ),
        out_specs=(
            pl.BlockSpec((N_ATOMS, HIDDEN_DIM), lambda: (0, 0)),
            pl.BlockSpec((N_ATOMS, NUM_FILTERS), lambda: (0, 0)),
            pl.BlockSpec((32, 128), lambda: (0, 0)),
            pl.BlockSpec((32, 128), lambda: (0, 0)),
            pl.BlockSpec(memory_space=pltpu.SMEM),
        ),
    )(x, ew, eb, l1w, l1b, counts)


def _w_body(nb_ref, d2_ref, f1t_ref, b1_ref, f2t_ref, b2_ref, w_ref):
    i = pl.program_id(0)

    @pl.when(i < nb_ref[0])
    def _():
        mu = lax.broadcasted_iota(jnp.int32, (NUM_GAUSSIANS, 1), 0).astype(
            jnp.float32) * (CUTOFF / (NUM_GAUSSIANS - 1))
        d2b = d2_ref[...]
        f1t = f1t_ref[...]
        b1 = b1_ref[...]
        f2t = f2t_ref[...]
        b2 = b2_ref[...]
        for r in range(8):
            d = jnp.sqrt(d2b[r:r + 1, :] + 1e-12)
            g = jnp.exp(GCOEF * (d - mu) ** 2)
            w1 = jnp.dot(f1t, g, preferred_element_type=jnp.float32) + b1
            w1 = _ssp(w1)
            w2 = jnp.dot(f2t, w1, preferred_element_type=jnp.float32) + b2
            w_ref[pl.ds(r * 128, 128), :] = w2.T


def _w_call(nblk, d2r, f1t, b1c, f2t, b2c):
    grid_spec = pltpu.PrefetchScalarGridSpec(
        num_scalar_prefetch=1,
        grid=(NBLK,),
        in_specs=[
            pl.BlockSpec((8, 128), lambda i, nb: (i, 0)),
            pl.BlockSpec((NUM_FILTERS, NUM_GAUSSIANS), lambda i, nb: (0, 0)),
            pl.BlockSpec((NUM_FILTERS, 1), lambda i, nb: (0, 0)),
            pl.BlockSpec((NUM_FILTERS, NUM_FILTERS), lambda i, nb: (0, 0)),
            pl.BlockSpec((NUM_FILTERS, 1), lambda i, nb: (0, 0)),
        ],
        out_specs=pl.BlockSpec((BLK, 128), lambda i, nb: (i, 0)),
    )
    return pl.pallas_call(
        _w_body,
        grid_spec=grid_spec,
        out_shape=_SDS((E_BUF, NUM_FILTERS), jnp.float32),
    )(nblk, d2r, f1t, b1c, f2t, b2c)


def _sc_agg_fn():
    @functools.partial(
        pl.kernel,
        out_type=_SDS((N_ATOMS, NUM_FILTERS), jnp.float32),
        mesh=_VMESH(),
        compiler_params=_SC_PARAMS,
        scratch_types=[
            pltpu.VMEM((144,), jnp.int32),
            pltpu.VMEM((144,), jnp.int32),
            pltpu.VMEM((64,), jnp.int32),
            pltpu.VMEM((64, NUM_FILTERS), jnp.float32),
            pltpu.VMEM((64, NUM_FILTERS), jnp.float32),
            pltpu.VMEM((APW, NUM_FILTERS), jnp.float32),
            pltpu.SemaphoreType.DMA,
        ],
    )
    def k(xl_hbm, w_hbm, col_hbm, off_hbm, cl_hbm, agg_hbm,
          offv, clv, colv, wv, xgv, aggv, sem):
        w = _wid()
        a0 = w * APW
        pltpu.sync_copy(off_hbm.at[pl.ds(a0, APW)], offv.at[pl.ds(0, APW)])
        pltpu.sync_copy(cl_hbm.at[pl.ds(a0, APW)], clv.at[pl.ds(0, APW)])

        def atom(i, carry):
            off = pl.multiple_of(_vscal(offv, i), 8)
            cnt = _vscal(clv, i)
            ngr = (cnt + 63) // 64

            def grp(g, acc):
                base = pl.multiple_of(off + g * 64, 8)
                pltpu.sync_copy(w_hbm.at[pl.ds(base, 64), :], wv)
                pltpu.sync_copy(col_hbm.at[pl.ds(base, 64)], colv)
                pltpu.async_copy(xl_hbm.at[colv], xgv, sem).wait()
                en = jnp.minimum(cnt - g * 64, 64)

                def edge(e, acc):
                    return tuple(
                        acc[f] + wv[e, pl.ds(f * 16, 16)]
                        * xgv[e, pl.ds(f * 16, 16)]
                        for f in range(8))
                return lax.fori_loop(0, en, edge, acc)

            acc0 = tuple(jnp.zeros((16,), jnp.float32) for _ in range(8))
            acc = lax.fori_loop(0, ngr, grp, acc0)
            for f in range(8):
                aggv[i, pl.ds(f * 16, 16)] = acc[f]
            return carry

        lax.fori_loop(0, APW, atom, jnp.int32(0))
        pltpu.sync_copy(aggv, agg_hbm.at[pl.ds(a0, APW), :])
    return k


def _post_body(h_ref, agg_ref, l2w_ref, l2b_ref, ow_ref, ob_ref,
               n1w_ref, n1b_ref, h_out, xl_out):
    hc = jnp.dot(agg_ref[...], l2w_ref[...],
                 preferred_element_type=jnp.float32) + l2b_ref[...]
    hc = _ssp(hc)
    hc = jnp.dot(hc, ow_ref[...],
                 preferred_element_type=jnp.float32) + ob_ref[...]
    hn = h_ref[...] + hc
    h_out[...] = hn
    xl_out[...] = jnp.dot(hn, n1w_ref[...],
                          preferred_element_type=jnp.float32) + n1b_ref[...]


def _post_call(h, agg, l2w, l2b, ow, ob, n1w, n1b):
    return pl.pallas_call(
        _post_body,
        out_shape=(_SDS((N_ATOMS, HIDDEN_DIM), jnp.float32),
                   _SDS((N_ATOMS, NUM_FILTERS), jnp.float32)),
    )(h, agg, l2w, l2b, ow, ob, n1w, n1b)


def _final_body(h_ref, agg_ref, l2w_ref, l2b_ref, ow_ref, ob_ref,
                batch_ref, w0_ref, b0_ref, w1_ref, b1_ref, w2_ref, b2_ref,
                out_ref):
    hc = jnp.dot(agg_ref[...], l2w_ref[...],
                 preferred_element_type=jnp.float32) + l2b_ref[...]
    hc = _ssp(hc)
    hc = jnp.dot(hc, ow_ref[...],
                 preferred_element_type=jnp.float32) + ob_ref[...]
    hn = h_ref[...] + hc
    batch = batch_ref[...]
    gids = lax.broadcasted_iota(jnp.int32, (NUM_GRAPHS, N_ATOMS), 0)
    onehot = (batch == gids).astype(jnp.float32)
    hg = jnp.dot(onehot, hn, preferred_element_type=jnp.float32)
    o = _ssp(jnp.dot(hg, w0_ref[...],
                     preferred_element_type=jnp.float32) + b0_ref[...])
    o = _ssp(jnp.dot(o, w1_ref[...],
                     preferred_element_type=jnp.float32) + b1_ref[...])
    o = jax.nn.sigmoid(jnp.dot(o, w2_ref[...],
                               preferred_element_type=jnp.float32) + b2_ref[...])
    out_ref[...] = o


def _final_call(h, agg, l2w, l2b, ow, ob, batch, p):
    return pl.pallas_call(
        _final_body,
        out_shape=_SDS((NUM_GRAPHS, OUTPUT_DIM), jnp.float32),
    )(h, agg, l2w, l2b, ow, ob, batch.reshape(1, N_ATOMS),
      p['mlp0'][0], p['mlp0'][1].reshape(1, -1),
      p['mlp1'][0], p['mlp1'][1].reshape(1, -1),
      p['mlp2'][0], p['mlp2'][1].reshape(1, -1))


def kernel(x, pos, batch, params):
    bi = batch.astype(jnp.int32)
    ss = jnp.searchsorted(bi, bi, side='left').astype(jnp.int32)
    se = jnp.searchsorted(bi, bi, side='right').astype(jnp.int32)
    px = pos[:, 0] + 0.0
    py = pos[:, 1] + 0.0
    pz = pos[:, 2] + 0.0

    counts = _sc_count_fn()(px, py, pz, ss, se)

    p0 = params['inter'][0]
    h, xlin, offc, clc, nblk = _emb_call(
        x, params['emb'][0], params['emb'][1].reshape(1, -1),
        p0['lin1'][0], p0['lin1'][1].reshape(1, -1),
        counts.reshape(32, 128))
    offc = offc.reshape(-1)
    clc = clc.reshape(-1)
    nblk = nblk.reshape(1)

    col, d2 = _sc_build_fn()(px, py, pz, ss, se, offc, clc)
    d2r = d2.reshape(E_BUF // 128, 128)

    sc_agg = _sc_agg_fn()
    for l in range(NUM_INTER):
        p = params['inter'][l]
        W = _w_call(nblk, d2r,
                    p['f1'][0].T, p['f1'][1].reshape(-1, 1),
                    p['f2'][0].T, p['f2'][1].reshape(-1, 1))
        agg = sc_agg(xlin, W, col, offc, clc)
        if l < NUM_INTER - 1:
            pn = params['inter'][l + 1]
            h, xlin = _post_call(
                h, agg, p['lin2'][0], p['lin2'][1].reshape(1, -1),
                p['out'][0], p['out'][1].reshape(1, -1),
                pn['lin1'][0], pn['lin1'][1].reshape(1, -1))
        else:
            out = _final_call(
                h, agg, p['lin2'][0], p['lin2'][1].reshape(1, -1),
                p['out'][0], p['out'][1].reshape(1, -1), bi, params)
    return out


# speedup vs baseline: 3.2653x; 1.0064x over previous
import functools

import jax
import jax.numpy as jnp
from jax import lax
from jax.experimental import pallas as pl
from jax.experimental.pallas import tpu as pltpu
from jax.experimental.pallas import tpu_sc as plsc

CUTOFF = 5.0
NUM_GAUSSIANS = 50
HIDDEN_DIM = 128
NUM_FILTERS = 128
NUM_INTER = 3
OUTPUT_DIM = 250
NUM_GRAPHS = 128
N_ATOMS = 4096
E_MAX = 262144
E_BUF = 263168
BLK = 1024
NBLK = E_BUF // BLK
NW = 32
APW = N_ATOMS // NW
STG = 2080
GCOEF = -0.5 / (CUTOFF / (NUM_GAUSSIANS - 1)) ** 2
LN2 = 0.6931471805599453

_SC_PARAMS = pltpu.CompilerParams(needs_layout_passes=False)
_VMESH = lambda: plsc.VectorSubcoreMesh(core_axis_name="c", subcore_axis_name="s")
_SDS = jax.ShapeDtypeStruct


def _ssp(x):
    return jnp.maximum(x, 0.0) + jnp.log1p(jnp.exp(-jnp.abs(x))) - LN2


def _wid():
    return lax.axis_index("s") * 2 + lax.axis_index("c")


def _vscal(ref, i):
    return ref[pl.ds(i, 16)][0]


def _popcnt(m):
    pc = plsc.all_reduce_population_count(m)
    return pc[0] if getattr(pc, "ndim", 0) else pc


def _pair_groups(pxv, pyv, pzv, i_glob, s, e):
    xi = jnp.full((16,), _vscal(pxv, i_glob), jnp.float32)
    yi = jnp.full((16,), _vscal(pyv, i_glob), jnp.float32)
    zi = jnp.full((16,), _vscal(pzv, i_glob), jnp.float32)
    base16 = (s // 16) * 16
    ng = (e - base16 + 15) // 16
    def group(g):
        j0 = base16 + g * 16
        jj = j0 + lax.iota(jnp.int32, 16)
        dx = pxv[pl.ds(j0, 16)] - xi
        dy = pyv[pl.ds(j0, 16)] - yi
        dz = pzv[pl.ds(j0, 16)] - zi
        d2 = dx * dx + dy * dy + dz * dz
        m = (jj >= s) & (jj < e) & (d2 < CUTOFF * CUTOFF) & (d2 > 1e-9)
        return jj, d2, m
    return ng, group


def _sc_count_fn():
    @functools.partial(
        pl.kernel,
        out_type=_SDS((N_ATOMS,), jnp.int32),
        mesh=_VMESH(),
        compiler_params=_SC_PARAMS,
        scratch_types=[
            pltpu.VMEM((4112,), jnp.float32),
            pltpu.VMEM((4112,), jnp.float32),
            pltpu.VMEM((4112,), jnp.float32),
            pltpu.VMEM((144,), jnp.int32),
            pltpu.VMEM((144,), jnp.int32),
            pltpu.VMEM((144,), jnp.int32),
        ],
    )
    def k(px_hbm, py_hbm, pz_hbm, ss_hbm, se_hbm, cnt_hbm,
          pxv, pyv, pzv, ssv, sev, cbv):
        w = _wid()
        a0 = w * APW
        pltpu.sync_copy(px_hbm, pxv.at[pl.ds(0, N_ATOMS)])
        pltpu.sync_copy(py_hbm, pyv.at[pl.ds(0, N_ATOMS)])
        pltpu.sync_copy(pz_hbm, pzv.at[pl.ds(0, N_ATOMS)])
        pltpu.sync_copy(ss_hbm.at[pl.ds(a0, APW)], ssv.at[pl.ds(0, APW)])
        pltpu.sync_copy(se_hbm.at[pl.ds(a0, APW)], sev.at[pl.ds(0, APW)])

        def atom(i, carry):
            s = _vscal(ssv, i)
            e = _vscal(sev, i)
            ng, group = _pair_groups(pxv, pyv, pzv, a0 + i, s, e)
            def body(g, cnt):
                _, _, m = group(g)
                return cnt + _popcnt(m)
            cnt = lax.fori_loop(0, ng, body, jnp.int32(0))
            tgt = jnp.where(lax.iota(jnp.int32, 16) == 0, i, 143)
            plsc.store_scatter(cbv, [tgt], jnp.full((16,), cnt, jnp.int32))
            return carry
        lax.fori_loop(0, APW, atom, jnp.int32(0))
        pltpu.sync_copy(cbv.at[pl.ds(0, APW)], cnt_hbm.at[pl.ds(a0, APW)])
    return k


def _sc_build_fn():
    @functools.partial(
        pl.kernel,
        out_type=(_SDS((E_BUF,), jnp.int32), _SDS((E_BUF,), jnp.float32)),
        mesh=_VMESH(),
        compiler_params=_SC_PARAMS,
        scratch_types=[
            pltpu.VMEM((4112,), jnp.float32),
            pltpu.VMEM((4112,), jnp.float32),
            pltpu.VMEM((4112,), jnp.float32),
            pltpu.VMEM((144,), jnp.int32),
            pltpu.VMEM((144,), jnp.int32),
            pltpu.VMEM((144,), jnp.int32),
            pltpu.VMEM((144,), jnp.int32),
            pltpu.VMEM((STG,), jnp.int32),
            pltpu.VMEM((STG,), jnp.float32),
        ],
    )
    def k(px_hbm, py_hbm, pz_hbm, ss_hbm, se_hbm, off_hbm, cl_hbm,
          col_hbm, d2_hbm,
          pxv, pyv, pzv, ssv, sev, offv, clv, scol, sd2):
        w = _wid()
        a0 = w * APW
        pltpu.sync_copy(px_hbm, pxv.at[pl.ds(0, N_ATOMS)])
        pltpu.sync_copy(py_hbm, pyv.at[pl.ds(0, N_ATOMS)])
        pltpu.sync_copy(pz_hbm, pzv.at[pl.ds(0, N_ATOMS)])
        pltpu.sync_copy(ss_hbm.at[pl.ds(a0, APW)], ssv.at[pl.ds(0, APW)])
        pltpu.sync_copy(se_hbm.at[pl.ds(a0, APW)], sev.at[pl.ds(0, APW)])
        pltpu.sync_copy(off_hbm.at[pl.ds(a0, APW)], offv.at[pl.ds(0, APW)])
        pltpu.sync_copy(cl_hbm.at[pl.ds(a0, APW)], clv.at[pl.ds(0, APW)])
        gbase = pl.multiple_of(_vscal(offv, 0), 8)

        def flush_if(spos, fl):
            def do(c):
                spos, fl = c
                ptr = pl.multiple_of(gbase + fl * 2048, 8)
                pltpu.sync_copy(scol.at[pl.ds(0, 2048)],
                                col_hbm.at[pl.ds(ptr, 2048)])
                pltpu.sync_copy(sd2.at[pl.ds(0, 2048)],
                                d2_hbm.at[pl.ds(ptr, 2048)])
                v0 = scol[pl.ds(2048, 16)]
                v1 = scol[pl.ds(2064, 16)]
                scol[pl.ds(0, 16)] = v0
                scol[pl.ds(16, 16)] = v1
                u0 = sd2[pl.ds(2048, 16)]
                u1 = sd2[pl.ds(2064, 16)]
                sd2[pl.ds(0, 16)] = u0
                sd2[pl.ds(16, 16)] = u1
                return (spos - 2048, fl + 1)
            return lax.cond(spos >= 2048, do, lambda c: c, (spos, fl))

        def atom(i, fl):
            s = _vscal(ssv, i)
            e = _vscal(sev, i)
            off = _vscal(offv, i)
            cl = _vscal(clv, i)
            spos0 = off - gbase - fl * 2048
            ng, group = _pair_groups(pxv, pyv, pzv, a0 + i, s, e)

            def grp(g, c):
                spos, app, fl = c
                jj, d2, m = group(g)
                mi = jnp.where(m, jnp.int32(1), jnp.int32(0))
                cs = plsc.cumsum(mi)
                m2 = m & ((cs + app) <= cl)
                mi2 = jnp.where(m2, jnp.int32(1), jnp.int32(0))
                cs2 = plsc.cumsum(mi2)
                tgt = jnp.where(m2, spos + cs2 - 1, jnp.int32(STG - 1))
                plsc.store_scatter(scol, [tgt], jj)
                plsc.store_scatter(sd2, [tgt], d2)
                pc = _popcnt(m2)
                spos, fl = flush_if(spos + pc, fl)
                return (spos, app + pc, fl)

            spos, app, fl = lax.fori_loop(0, ng, grp, (spos0, jnp.int32(0), fl))
            scol[pl.ds(spos, 16)] = jnp.zeros((16,), jnp.int32)
            sd2[pl.ds(spos, 16)] = jnp.zeros((16,), jnp.float32)
            spos, fl = flush_if(spos, fl)
            return fl

        fl = lax.fori_loop(0, APW, atom, jnp.int32(0))

        off_l = _vscal(offv, APW - 1)
        cl_l = _vscal(clv, APW - 1)
        end = off_l + ((cl_l + 7) // 8) * 8
        rem = end - gbase - fl * 2048
        ptr = gbase + fl * 2048
        n64 = rem // 64
        def f64(i, _):
            p = pl.multiple_of(ptr + i * 64, 8)
            pltpu.sync_copy(scol.at[pl.ds(i * 64, 64)], col_hbm.at[pl.ds(p, 64)])
            pltpu.sync_copy(sd2.at[pl.ds(i * 64, 64)], d2_hbm.at[pl.ds(p, 64)])
            return _
        lax.fori_loop(0, n64, f64, jnp.int32(0))
        t0 = n64 * 64
        n8 = (rem - t0) // 8
        def f8(i, _):
            p = pl.multiple_of(ptr + t0 + i * 8, 8)
            pltpu.sync_copy(scol.at[pl.ds(t0 + i * 8, 8)], col_hbm.at[pl.ds(p, 8)])
            pltpu.sync_copy(sd2.at[pl.ds(t0 + i * 8, 8)], d2_hbm.at[pl.ds(p, 8)])
            return _
        lax.fori_loop(0, n8, f8, jnp.int32(0))

        @pl.when(w == NW - 1)
        def _():
            scol[pl.ds(0, 16)] = jnp.zeros((16,), jnp.int32)
            scol[pl.ds(16, 16)] = jnp.zeros((16,), jnp.int32)
            scol[pl.ds(32, 16)] = jnp.zeros((16,), jnp.int32)
            scol[pl.ds(48, 16)] = jnp.zeros((16,), jnp.int32)
            p = pl.multiple_of(end, 8)
            pltpu.sync_copy(scol.at[pl.ds(0, 64)], col_hbm.at[pl.ds(p, 64)])
    return k


def _emb_body(x_ref, ew_ref, eb_ref, l1w_ref, l1b_ref, c_ref,
              h_ref, xl_ref, off_ref, cl_ref, nb_ref):
    h = jnp.dot(x_ref[...], ew_ref[...], preferred_element_type=jnp.float32)
    h = h + eb_ref[...]
    h_ref[...] = h
    xl_ref[...] = jnp.dot(h, l1w_ref[...],
                          preferred_element_type=jnp.float32) + l1b_ref[...]

    c = c_ref[...]
    c8 = jnp.bitwise_and(c + 7, -8)
    x = c8
    for sft in (1, 2, 4, 8, 16, 32, 64):
        sh = jnp.concatenate(
            [jnp.zeros((32, sft), jnp.int32), x[:, :128 - sft]], axis=1)
        x = x + sh
    ci = x
    row_tot = ci[:, 127:128]
    y = row_tot
    for sft in (1, 2, 4, 8, 16):
        sh = jnp.concatenate(
            [jnp.zeros((sft, 1), jnp.int32), y[:32 - sft, :]], axis=0)
        y = y + sh
    rb = y - row_tot
    off_excl = ci + rb - c8
    offc = jnp.minimum(off_excl, E_MAX)
    clc = jnp.minimum(c, jnp.maximum(E_MAX - offc, 0))
    off_ref[...] = offc
    cl_ref[...] = clc
    total8 = jnp.minimum(rb[31, 0] + row_tot[31, 0], E_MAX + 8)
    nb_ref[0, 0] = (total8 + BLK - 1) // BLK


def _emb_call(x, ew, eb, l1w, l1b, counts):
    return pl.pallas_call(
        _emb_body,
        out_shape=(
            _SDS((N_ATOMS, HIDDEN_DIM), jnp.float32),
            _SDS((N_ATOMS, NUM_FILTERS), jnp.float32),
            _SDS((32, 128), jnp.int32),
            _SDS((32, 128), jnp.int32),
            _SDS((1, 1), jnp.int32),
        ),
        out_specs=(
            pl.BlockSpec((N_ATOMS, HIDDEN_DIM), lambda: (0, 0)),
            pl.BlockSpec((N_ATOMS, NUM_FILTERS), lambda: (0, 0)),
            pl.BlockSpec((32, 128), lambda: (0, 0)),
            pl.BlockSpec((32, 128), lambda: (0, 0)),
            pl.BlockSpec(memory_space=pltpu.SMEM),
        ),
    )(x, ew, eb, l1w, l1b, counts)


def _w_body(nb_ref, d2_ref, f1t_ref, b1_ref, f2t_ref, b2_ref, w_ref):
    i = pl.program_id(0)

    @pl.when(i < nb_ref[0])
    def _():
        mu = lax.broadcasted_iota(jnp.int32, (NUM_GAUSSIANS, 1), 0).astype(
            jnp.float32) * (CUTOFF / (NUM_GAUSSIANS - 1))
        d2b = d2_ref[...]
        f1t = f1t_ref[...]
        b1 = b1_ref[...]
        f2t = f2t_ref[...]
        b2 = b2_ref[...]
        for r in range(8):
            d = jnp.sqrt(d2b[r:r + 1, :] + 1e-12)
            g = jnp.exp(GCOEF * (d - mu) ** 2)
            w1 = jnp.dot(f1t, g, preferred_element_type=jnp.float32) + b1
            w1 = _ssp(w1)
            w2 = jnp.dot(f2t, w1, preferred_element_type=jnp.float32) + b2
            w_ref[pl.ds(r * 128, 128), :] = w2.T


def _w_call(nblk, d2r, f1t, b1c, f2t, b2c):
    grid_spec = pltpu.PrefetchScalarGridSpec(
        num_scalar_prefetch=1,
        grid=(NBLK,),
        in_specs=[
            pl.BlockSpec((8, 128), lambda i, nb: (i, 0)),
            pl.BlockSpec((NUM_FILTERS, NUM_GAUSSIANS), lambda i, nb: (0, 0)),
            pl.BlockSpec((NUM_FILTERS, 1), lambda i, nb: (0, 0)),
            pl.BlockSpec((NUM_FILTERS, NUM_FILTERS), lambda i, nb: (0, 0)),
            pl.BlockSpec((NUM_FILTERS, 1), lambda i, nb: (0, 0)),
        ],
        out_specs=pl.BlockSpec((BLK, 128), lambda i, nb: (i, 0)),
    )
    return pl.pallas_call(
        _w_body,
        grid_spec=grid_spec,
        out_shape=_SDS((E_BUF, NUM_FILTERS), jnp.float32),
    )(nblk, d2r, f1t, b1c, f2t, b2c)


def _sc_agg_fn():
    @functools.partial(
        pl.kernel,
        out_type=_SDS((N_ATOMS, NUM_FILTERS), jnp.float32),
        mesh=_VMESH(),
        compiler_params=_SC_PARAMS,
        scratch_types=[
            pltpu.VMEM_SHARED((N_ATOMS, NUM_FILTERS), jnp.float32),
            pltpu.VMEM((144,), jnp.int32),
            pltpu.VMEM((144,), jnp.int32),
            pltpu.VMEM((64,), jnp.int32),
            pltpu.VMEM((64,), jnp.int32),
            pltpu.VMEM((64,), jnp.int32),
            pltpu.VMEM((64, NUM_FILTERS), jnp.float32),
            pltpu.VMEM((64, NUM_FILTERS), jnp.float32),
            pltpu.VMEM((64, NUM_FILTERS), jnp.float32),
            pltpu.VMEM((64, NUM_FILTERS), jnp.float32),
            pltpu.VMEM((APW, NUM_FILTERS), jnp.float32),
            pltpu.SemaphoreType.DMA,
            pltpu.SemaphoreType.DMA,
            pltpu.SemaphoreType.DMA,
            pltpu.SemaphoreType.DMA,
            pltpu.SemaphoreType.DMA,
            pltpu.SemaphoreType.DMA,
            pltpu.SemaphoreType.DMA,
        ],
    )
    def k(xl_hbm, w_hbm, col_hbm, off_hbm, cl_hbm, agg_hbm,
          shv, offv, clv, cb0, cb1, cslow, wb0, wb1, xb0, xb1, aggv,
          sw0, sw1, sx0, sx1, sc0, sc1, sslow):
        w = _wid()
        a0 = w * APW
        sid = lax.axis_index("s")

        @pl.when(sid == 0)
        def _stage():
            pltpu.sync_copy(xl_hbm, shv)
        pltpu.sync_copy(off_hbm.at[pl.ds(a0, APW)], offv.at[pl.ds(0, APW)])
        pltpu.sync_copy(cl_hbm.at[pl.ds(a0, APW)], clv.at[pl.ds(0, APW)])
        plsc.subcore_barrier()

        off0 = pl.multiple_of(_vscal(offv, 0), 8)
        pltpu.async_copy(col_hbm.at[pl.ds(off0, 64)], cb0, sc0).wait()

        def step(i, cb_c, wb_c, xb_c, sw_c, sx_c, cb_p, wb_p, xb_p,
                 sw_p, sx_p, sc_p):
            ii = jnp.minimum(i, APW - 1)
            off_i = pl.multiple_of(_vscal(offv, ii), 8)
            inx = jnp.minimum(i + 1, APW - 1)
            off_n = pl.multiple_of(_vscal(offv, inx), 8)

            @pl.when(i < APW)
            def _issue():
                pltpu.async_copy(w_hbm.at[pl.ds(off_i, 64), :], wb_c, sw_c)
                pltpu.async_copy(xl_hbm.at[cb_c], xb_c, sx_c)

                @pl.when(i + 1 < APW)
                def _pref_col():
                    pltpu.async_copy(col_hbm.at[pl.ds(off_n, 64)], cb_p, sc_p)

            @pl.when(i >= 1)
            def _proc():
                j = i - 1
                cnt = _vscal(clv, j)
                en0 = jnp.minimum(cnt, 64)

                def edge(e, acc):
                    return tuple(
                        acc[f] + wb_p[e, pl.ds(f * 16, 16)]
                        * xb_p[e, pl.ds(f * 16, 16)]
                        for f in range(8))
                acc0 = tuple(jnp.zeros((16,), jnp.float32) for _ in range(8))
                acc = lax.fori_loop(0, en0, edge, acc0)

                ngr = (cnt + 63) // 64

                def slow(g, acc):
                    off_j = pl.multiple_of(_vscal(offv, j), 8)
                    base = pl.multiple_of(off_j + g * 64, 8)
                    pltpu.sync_copy(w_hbm.at[pl.ds(base, 64), :], wb_p)
                    pltpu.sync_copy(col_hbm.at[pl.ds(base, 64)], cslow)
                    pltpu.async_copy(xl_hbm.at[cslow], xb_p, sslow).wait()
                    en = jnp.minimum(cnt - g * 64, 64)
                    return lax.fori_loop(0, en, edge, acc)
                acc = lax.fori_loop(1, ngr, slow, acc)
                for f in range(8):
                    aggv[j, pl.ds(f * 16, 16)] = acc[f]

            @pl.when(i < APW)
            def _finwait():
                pltpu.make_async_copy(
                    w_hbm.at[pl.ds(off_i, 64), :], wb_c, sw_c).wait()
                pltpu.make_async_copy(xl_hbm.at[cb_c], xb_c, sx_c).wait()

                @pl.when(i + 1 < APW)
                def _wait_col():
                    pltpu.make_async_copy(
                        col_hbm.at[pl.ds(off_n, 64)], cb_p, sc_p).wait()

        def body(i, carry):
            lax.cond(
                i % 2 == 0,
                lambda: step(i, cb0, wb0, xb0, sw0, sx0,
                             cb1, wb1, xb1, sw1, sx1, sc1),
                lambda: step(i, cb1, wb1, xb1, sw1, sx1,
                             cb0, wb0, xb0, sw0, sx0, sc0))
            return carry
        lax.fori_loop(0, APW + 1, body, jnp.int32(0))
        pltpu.sync_copy(aggv, agg_hbm.at[pl.ds(a0, APW), :])
    return k


def _post_body(h_ref, agg_ref, l2w_ref, l2b_ref, ow_ref, ob_ref,
               n1w_ref, n1b_ref, h_out, xl_out):
    hc = jnp.dot(agg_ref[...], l2w_ref[...],
                 preferred_element_type=jnp.float32) + l2b_ref[...]
    hc = _ssp(hc)
    hc = jnp.dot(hc, ow_ref[...],
                 preferred_element_type=jnp.float32) + ob_ref[...]
    hn = h_ref[...] + hc
    h_out[...] = hn
    xl_out[...] = jnp.dot(hn, n1w_ref[...],
                          preferred_element_type=jnp.float32) + n1b_ref[...]


def _post_call(h, agg, l2w, l2b, ow, ob, n1w, n1b):
    return pl.pallas_call(
        _post_body,
        out_shape=(_SDS((N_ATOMS, HIDDEN_DIM), jnp.float32),
                   _SDS((N_ATOMS, NUM_FILTERS), jnp.float32)),
    )(h, agg, l2w, l2b, ow, ob, n1w, n1b)


def _final_body(h_ref, agg_ref, l2w_ref, l2b_ref, ow_ref, ob_ref,
                batch_ref, w0_ref, b0_ref, w1_ref, b1_ref, w2_ref, b2_ref,
                out_ref):
    hc = jnp.dot(agg_ref[...], l2w_ref[...],
                 preferred_element_type=jnp.float32) + l2b_ref[...]
    hc = _ssp(hc)
    hc = jnp.dot(hc, ow_ref[...],
                 preferred_element_type=jnp.float32) + ob_ref[...]
    hn = h_ref[...] + hc
    batch = batch_ref[...]
    gids = lax.broadcasted_iota(jnp.int32, (NUM_GRAPHS, N_ATOMS), 0)
    onehot = (batch == gids).astype(jnp.float32)
    hg = jnp.dot(onehot, hn, preferred_element_type=jnp.float32)
    o = _ssp(jnp.dot(hg, w0_ref[...],
                     preferred_element_type=jnp.float32) + b0_ref[...])
    o = _ssp(jnp.dot(o, w1_ref[...],
                     preferred_element_type=jnp.float32) + b1_ref[...])
    o = jax.nn.sigmoid(jnp.dot(o, w2_ref[...],
                               preferred_element_type=jnp.float32) + b2_ref[...])
    out_ref[...] = o


def _final_call(h, agg, l2w, l2b, ow, ob, batch, p):
    return pl.pallas_call(
        _final_body,
        out_shape=_SDS((NUM_GRAPHS, OUTPUT_DIM), jnp.float32),
    )(h, agg, l2w, l2b, ow, ob, batch.reshape(1, N_ATOMS),
      p['mlp0'][0], p['mlp0'][1].reshape(1, -1),
      p['mlp1'][0], p['mlp1'][1].reshape(1, -1),
      p['mlp2'][0], p['mlp2'][1].reshape(1, -1))


def kernel(x, pos, batch, params):
    bi = batch.astype(jnp.int32)
    ss = jnp.searchsorted(bi, bi, side='left').astype(jnp.int32)
    se = jnp.searchsorted(bi, bi, side='right').astype(jnp.int32)
    px = pos[:, 0] + 0.0
    py = pos[:, 1] + 0.0
    pz = pos[:, 2] + 0.0

    counts = _sc_count_fn()(px, py, pz, ss, se)

    p0 = params['inter'][0]
    h, xlin, offc, clc, nblk = _emb_call(
        x, params['emb'][0], params['emb'][1].reshape(1, -1),
        p0['lin1'][0], p0['lin1'][1].reshape(1, -1),
        counts.reshape(32, 128))
    offc = offc.reshape(-1)
    clc = clc.reshape(-1)
    nblk = nblk.reshape(1)

    col, d2 = _sc_build_fn()(px, py, pz, ss, se, offc, clc)
    d2r = d2.reshape(E_BUF // 128, 128)

    sc_agg = _sc_agg_fn()
    for l in range(NUM_INTER):
        p = params['inter'][l]
        W = _w_call(nblk, d2r,
                    p['f1'][0].T, p['f1'][1].reshape(-1, 1),
                    p['f2'][0].T, p['f2'][1].reshape(-1, 1))
        agg = sc_agg(xlin, W, col, offc, clc)
        if l < NUM_INTER - 1:
            pn = params['inter'][l + 1]
            h, xlin = _post_call(
                h, agg, p['lin2'][0], p['lin2'][1].reshape(1, -1),
                p['out'][0], p['out'][1].reshape(1, -1),
                pn['lin1'][0], pn['lin1'][1].reshape(1, -1))
        else:
            out = _final_call(
                h, agg, p['lin2'][0], p['lin2'][1].reshape(1, -1),
                p['out'][0], p['out'][1].reshape(1, -1), bi, params)
    return out


# speedup vs baseline: 10.3226x; 3.1613x over previous
import functools

import jax
import jax.numpy as jnp
from jax import lax
from jax.experimental import pallas as pl
from jax.experimental.pallas import tpu as pltpu
from jax.experimental.pallas import tpu_sc as plsc

CUTOFF = 5.0
NUM_GAUSSIANS = 50
HIDDEN_DIM = 128
NUM_FILTERS = 128
NUM_INTER = 3
OUTPUT_DIM = 250
NUM_GRAPHS = 128
N_ATOMS = 4096
E_MAX = 262144
E_BUF = 263168
BLK = 1024
NBLK = E_BUF // BLK
NW = 32
APW = N_ATOMS // NW
WIN = 512
STG = 2080
GCOEF = -0.5 / (CUTOFF / (NUM_GAUSSIANS - 1)) ** 2
LN2 = 0.6931471805599453

_SC_PARAMS = pltpu.CompilerParams(needs_layout_passes=False)
_VMESH = lambda: plsc.VectorSubcoreMesh(core_axis_name="c", subcore_axis_name="s")
_SDS = jax.ShapeDtypeStruct


def _ssp(x):
    return jnp.maximum(x, 0.0) + jnp.log1p(jnp.exp(-jnp.abs(x))) - LN2


def _wid():
    return lax.axis_index("s") * 2 + lax.axis_index("c")


def _vscal(ref, i):
    return ref[pl.ds(i, 16)][0]


def _popcnt(m):
    pc = plsc.all_reduce_population_count(m)
    return pc[0] if getattr(pc, "ndim", 0) else pc


def _pair_groups(pxv, pyv, pzv, i_glob, s, e):
    xi = jnp.full((16,), _vscal(pxv, i_glob), jnp.float32)
    yi = jnp.full((16,), _vscal(pyv, i_glob), jnp.float32)
    zi = jnp.full((16,), _vscal(pzv, i_glob), jnp.float32)
    base16 = (s // 16) * 16
    ng = (e - base16 + 15) // 16
    def group(g):
        j0 = base16 + g * 16
        jj = j0 + lax.iota(jnp.int32, 16)
        dx = pxv[pl.ds(j0, 16)] - xi
        dy = pyv[pl.ds(j0, 16)] - yi
        dz = pzv[pl.ds(j0, 16)] - zi
        d2 = dx * dx + dy * dy + dz * dz
        m = (jj >= s) & (jj < e) & (d2 < CUTOFF * CUTOFF) & (d2 > 1e-9)
        return jj, d2, m
    return ng, group


def _sc_count_fn():
    @functools.partial(
        pl.kernel,
        out_type=_SDS((N_ATOMS,), jnp.int32),
        mesh=_VMESH(),
        compiler_params=_SC_PARAMS,
        scratch_types=[
            pltpu.VMEM((4112,), jnp.float32),
            pltpu.VMEM((4112,), jnp.float32),
            pltpu.VMEM((4112,), jnp.float32),
            pltpu.VMEM((144,), jnp.int32),
            pltpu.VMEM((144,), jnp.int32),
            pltpu.VMEM((144,), jnp.int32),
        ],
    )
    def k(px_hbm, py_hbm, pz_hbm, ss_hbm, se_hbm, cnt_hbm,
          pxv, pyv, pzv, ssv, sev, cbv):
        w = _wid()
        a0 = w * APW
        pltpu.sync_copy(px_hbm, pxv.at[pl.ds(0, N_ATOMS)])
        pltpu.sync_copy(py_hbm, pyv.at[pl.ds(0, N_ATOMS)])
        pltpu.sync_copy(pz_hbm, pzv.at[pl.ds(0, N_ATOMS)])
        pltpu.sync_copy(ss_hbm.at[pl.ds(a0, APW)], ssv.at[pl.ds(0, APW)])
        pltpu.sync_copy(se_hbm.at[pl.ds(a0, APW)], sev.at[pl.ds(0, APW)])

        def atom(i, carry):
            s = _vscal(ssv, i)
            e = _vscal(sev, i)
            ng, group = _pair_groups(pxv, pyv, pzv, a0 + i, s, e)
            def body(g, cnt):
                _, _, m = group(g)
                return cnt + _popcnt(m)
            cnt = lax.fori_loop(0, ng, body, jnp.int32(0))
            tgt = jnp.where(lax.iota(jnp.int32, 16) == 0, i, 143)
            plsc.store_scatter(cbv, [tgt], jnp.full((16,), cnt, jnp.int32))
            return carry
        lax.fori_loop(0, APW, atom, jnp.int32(0))
        pltpu.sync_copy(cbv.at[pl.ds(0, APW)], cnt_hbm.at[pl.ds(a0, APW)])
    return k


def _sc_build_fn():
    @functools.partial(
        pl.kernel,
        out_type=(_SDS((E_BUF,), jnp.int32), _SDS((E_BUF,), jnp.float32)),
        mesh=_VMESH(),
        compiler_params=_SC_PARAMS,
        scratch_types=[
            pltpu.VMEM((4112,), jnp.float32),
            pltpu.VMEM((4112,), jnp.float32),
            pltpu.VMEM((4112,), jnp.float32),
            pltpu.VMEM((144,), jnp.int32),
            pltpu.VMEM((144,), jnp.int32),
            pltpu.VMEM((144,), jnp.int32),
            pltpu.VMEM((144,), jnp.int32),
            pltpu.VMEM((STG,), jnp.int32),
            pltpu.VMEM((STG,), jnp.float32),
        ],
    )
    def k(px_hbm, py_hbm, pz_hbm, ss_hbm, se_hbm, off_hbm, cl_hbm,
          col_hbm, d2_hbm,
          pxv, pyv, pzv, ssv, sev, offv, clv, scol, sd2):
        w = _wid()
        a0 = w * APW
        pltpu.sync_copy(px_hbm, pxv.at[pl.ds(0, N_ATOMS)])
        pltpu.sync_copy(py_hbm, pyv.at[pl.ds(0, N_ATOMS)])
        pltpu.sync_copy(pz_hbm, pzv.at[pl.ds(0, N_ATOMS)])
        pltpu.sync_copy(ss_hbm.at[pl.ds(a0, APW)], ssv.at[pl.ds(0, APW)])
        pltpu.sync_copy(se_hbm.at[pl.ds(a0, APW)], sev.at[pl.ds(0, APW)])
        pltpu.sync_copy(off_hbm.at[pl.ds(a0, APW)], offv.at[pl.ds(0, APW)])
        pltpu.sync_copy(cl_hbm.at[pl.ds(a0, APW)], clv.at[pl.ds(0, APW)])
        gbase = pl.multiple_of(_vscal(offv, 0), 8)

        def flush_if(spos, fl):
            def do(c):
                spos, fl = c
                ptr = pl.multiple_of(gbase + fl * 2048, 8)
                pltpu.sync_copy(scol.at[pl.ds(0, 2048)],
                                col_hbm.at[pl.ds(ptr, 2048)])
                pltpu.sync_copy(sd2.at[pl.ds(0, 2048)],
                                d2_hbm.at[pl.ds(ptr, 2048)])
                v0 = scol[pl.ds(2048, 16)]
                v1 = scol[pl.ds(2064, 16)]
                scol[pl.ds(0, 16)] = v0
                scol[pl.ds(16, 16)] = v1
                u0 = sd2[pl.ds(2048, 16)]
                u1 = sd2[pl.ds(2064, 16)]
                sd2[pl.ds(0, 16)] = u0
                sd2[pl.ds(16, 16)] = u1
                return (spos - 2048, fl + 1)
            return lax.cond(spos >= 2048, do, lambda c: c, (spos, fl))

        def atom(i, fl):
            s = _vscal(ssv, i)
            e = _vscal(sev, i)
            off = _vscal(offv, i)
            cl = _vscal(clv, i)
            spos0 = off - gbase - fl * 2048
            ng, group = _pair_groups(pxv, pyv, pzv, a0 + i, s, e)

            def grp(g, c):
                spos, app, fl = c
                jj, d2, m = group(g)
                mi = jnp.where(m, jnp.int32(1), jnp.int32(0))
                cs = plsc.cumsum(mi)
                m2 = m & ((cs + app) <= cl)
                mi2 = jnp.where(m2, jnp.int32(1), jnp.int32(0))
                cs2 = plsc.cumsum(mi2)
                tgt = jnp.where(m2, spos + cs2 - 1, jnp.int32(STG - 1))
                plsc.store_scatter(scol, [tgt], jj)
                plsc.store_scatter(sd2, [tgt], d2)
                pc = _popcnt(m2)
                spos, fl = flush_if(spos + pc, fl)
                return (spos, app + pc, fl)

            spos, app, fl = lax.fori_loop(0, ng, grp, (spos0, jnp.int32(0), fl))
            scol[pl.ds(spos, 16)] = jnp.full((16,), a0 + i, jnp.int32)
            sd2[pl.ds(spos, 16)] = jnp.zeros((16,), jnp.float32)
            spos, fl = flush_if(spos, fl)
            return fl

        fl = lax.fori_loop(0, APW, atom, jnp.int32(0))

        off_l = _vscal(offv, APW - 1)
        cl_l = _vscal(clv, APW - 1)
        end = off_l + ((cl_l + 7) // 8) * 8
        rem = end - gbase - fl * 2048
        ptr = gbase + fl * 2048
        n64 = rem // 64
        def f64(i, _):
            p = pl.multiple_of(ptr + i * 64, 8)
            pltpu.sync_copy(scol.at[pl.ds(i * 64, 64)], col_hbm.at[pl.ds(p, 64)])
            pltpu.sync_copy(sd2.at[pl.ds(i * 64, 64)], d2_hbm.at[pl.ds(p, 64)])
            return _
        lax.fori_loop(0, n64, f64, jnp.int32(0))
        t0 = n64 * 64
        n8 = (rem - t0) // 8
        def f8(i, _):
            p = pl.multiple_of(ptr + t0 + i * 8, 8)
            pltpu.sync_copy(scol.at[pl.ds(t0 + i * 8, 8)], col_hbm.at[pl.ds(p, 8)])
            pltpu.sync_copy(sd2.at[pl.ds(t0 + i * 8, 8)], d2_hbm.at[pl.ds(p, 8)])
            return _
        lax.fori_loop(0, n8, f8, jnp.int32(0))

        @pl.when(w == NW - 1)
        def _():
            it16 = lax.iota(jnp.int32, 16)
            scol[pl.ds(0, 16)] = it16
            scol[pl.ds(16, 16)] = it16 + 16
            scol[pl.ds(32, 16)] = it16 + 32
            scol[pl.ds(48, 16)] = it16 + 48
            p = pl.multiple_of(end, 8)
            pltpu.sync_copy(scol.at[pl.ds(0, 64)], col_hbm.at[pl.ds(p, 64)])
    return k


def _emb_body(x_ref, ew_ref, eb_ref, l1w_ref, l1b_ref, c_ref,
              h_ref, xl_ref, off_ref, cl_ref, nb_ref):
    h = jnp.dot(x_ref[...], ew_ref[...], preferred_element_type=jnp.float32)
    h = h + eb_ref[...]
    h_ref[...] = h
    xl_ref[...] = jnp.dot(h, l1w_ref[...],
                          preferred_element_type=jnp.float32) + l1b_ref[...]

    c = c_ref[...]
    c8 = jnp.bitwise_and(c + 7, -8)
    x = c8
    for sft in (1, 2, 4, 8, 16, 32, 64):
        sh = jnp.concatenate(
            [jnp.zeros((32, sft), jnp.int32), x[:, :128 - sft]], axis=1)
        x = x + sh
    ci = x
    row_tot = ci[:, 127:128]
    y = row_tot
    for sft in (1, 2, 4, 8, 16):
        sh = jnp.concatenate(
            [jnp.zeros((sft, 1), jnp.int32), y[:32 - sft, :]], axis=0)
        y = y + sh
    rb = y - row_tot
    off_excl = ci + rb - c8
    offc = jnp.minimum(off_excl, E_MAX)
    clc = jnp.minimum(c, jnp.maximum(E_MAX - offc, 0))
    off_ref[...] = offc
    cl_ref[...] = clc
    total8 = jnp.minimum(rb[31, 0] + row_tot[31, 0], E_MAX + 8)
    nb_ref[0, 0] = (total8 + BLK - 1) // BLK


def _emb_call(x, ew, eb, l1w, l1b, counts):
    return pl.pallas_call(
        _emb_body,
        out_shape=(
            _SDS((N_ATOMS, HIDDEN_DIM), jnp.float32),
            _SDS((N_ATOMS, NUM_FILTERS), jnp.float32),
            _SDS((32, 128), jnp.int32),
            _SDS((32, 128), jnp.int32),
            _SDS((1, 1), jnp.int32),
        ),
        out_specs=(
            pl.BlockSpec((N_ATOMS, HIDDEN_DIM), lambda: (0, 0)),
            pl.BlockSpec((N_ATOMS, NUM_FILTERS), lambda: (0, 0)),
            pl.BlockSpec((32, 128), lambda: (0, 0)),
            pl.BlockSpec((32, 128), lambda: (0, 0)),
            pl.BlockSpec(memory_space=pltpu.SMEM),
        ),
    )(x, ew, eb, l1w, l1b, counts)


def _w_body(nb_ref, d2_ref, f1t_ref, b1_ref, f2t_ref, b2_ref, w_ref):
    i = pl.program_id(0)

    @pl.when(i < nb_ref[0])
    def _():
        mu = lax.broadcasted_iota(jnp.int32, (NUM_GAUSSIANS, 1), 0).astype(
            jnp.float32) * (CUTOFF / (NUM_GAUSSIANS - 1))
        d2b = d2_ref[...]
        f1t = f1t_ref[...]
        b1 = b1_ref[...]
        f2t = f2t_ref[...]
        b2 = b2_ref[...]
        for r in range(8):
            d = jnp.sqrt(d2b[r:r + 1, :] + 1e-12)
            g = jnp.exp(GCOEF * (d - mu) ** 2)
            w1 = jnp.dot(f1t, g, preferred_element_type=jnp.float32) + b1
            w1 = _ssp(w1)
            w2 = jnp.dot(f2t, w1, preferred_element_type=jnp.float32) + b2
            w_ref[pl.ds(r * 128, 128), :] = w2.T


def _w_call(nblk, d2r, f1t, b1c, f2t, b2c):
    grid_spec = pltpu.PrefetchScalarGridSpec(
        num_scalar_prefetch=1,
        grid=(NBLK,),
        in_specs=[
            pl.BlockSpec((8, 128), lambda i, nb: (i, 0)),
            pl.BlockSpec((NUM_FILTERS, NUM_GAUSSIANS), lambda i, nb: (0, 0)),
            pl.BlockSpec((NUM_FILTERS, 1), lambda i, nb: (0, 0)),
            pl.BlockSpec((NUM_FILTERS, NUM_FILTERS), lambda i, nb: (0, 0)),
            pl.BlockSpec((NUM_FILTERS, 1), lambda i, nb: (0, 0)),
        ],
        out_specs=pl.BlockSpec((BLK, 128), lambda i, nb: (i, 0)),
    )
    return pl.pallas_call(
        _w_body,
        grid_spec=grid_spec,
        out_shape=_SDS((E_BUF, NUM_FILTERS), jnp.float32),
    )(nblk, d2r, f1t, b1c, f2t, b2c)


def _sc_agg_fn():
    @functools.partial(
        pl.kernel,
        out_type=_SDS((N_ATOMS, NUM_FILTERS), jnp.float32),
        mesh=_VMESH(),
        compiler_params=_SC_PARAMS,
        scratch_types=[
            pltpu.VMEM((WIN, NUM_FILTERS), jnp.float32),
            pltpu.VMEM((144,), jnp.int32),
            pltpu.VMEM((144,), jnp.int32),
            pltpu.VMEM((16,), jnp.int32),
            pltpu.VMEM((16,), jnp.int32),
            pltpu.VMEM((80,), jnp.int32),
            pltpu.VMEM((80,), jnp.int32),
            pltpu.VMEM((80,), jnp.int32),
            pltpu.VMEM((64, NUM_FILTERS), jnp.float32),
            pltpu.VMEM((64, NUM_FILTERS), jnp.float32),
            pltpu.VMEM((64, NUM_FILTERS), jnp.float32),
            pltpu.VMEM((64, NUM_FILTERS), jnp.float32),
            pltpu.VMEM((APW, NUM_FILTERS), jnp.float32),
            pltpu.SemaphoreType.DMA,
            pltpu.SemaphoreType.DMA,
            pltpu.SemaphoreType.DMA,
            pltpu.SemaphoreType.DMA,
            pltpu.SemaphoreType.DMA,
            pltpu.SemaphoreType.DMA,
            pltpu.SemaphoreType.DMA,
        ],
    )
    def k(xl_hbm, w_hbm, col_hbm, off_hbm, cl_hbm, ss_hbm, se_hbm, agg_hbm,
          xwin, offv, clv, jlov, jhiv, cb0, cb1, cslow, wb0, wb1, xb0, xb1,
          aggv, sw0, sw1, sx0, sx1, sc0, sc1, sslow):
        w = _wid()
        a0 = w * APW
        pltpu.sync_copy(off_hbm.at[pl.ds(a0, APW)], offv.at[pl.ds(0, APW)])
        pltpu.sync_copy(cl_hbm.at[pl.ds(a0, APW)], clv.at[pl.ds(0, APW)])
        pltpu.sync_copy(ss_hbm.at[pl.ds(a0, 16)], jlov)
        pltpu.sync_copy(se_hbm.at[pl.ds(a0 + APW - 16, 16)], jhiv)
        jlo = jlov[...][0]
        jhi = jhiv[...][15]
        jlo_eff = pl.multiple_of(
            jnp.minimum(jnp.bitwise_and(jlo, -8), N_ATOMS - WIN), 8)
        win_ok = (jhi - jlo_eff) <= WIN

        @pl.when(win_ok)
        def _stage_window():
            pltpu.sync_copy(xl_hbm.at[pl.ds(jlo_eff, WIN), :], xwin)

        off0 = pl.multiple_of(_vscal(offv, 0), 8)
        pltpu.async_copy(col_hbm.at[pl.ds(off0, 64)], cb0.at[pl.ds(0, 64)],
                         sc0).wait()

        def step(i, cb_c, wb_c, xb_c, sw_c, sx_c, cb_p, wb_p, xb_p,
                 sw_p, sx_p, sc_p):
            ii = jnp.minimum(i, APW - 1)
            off_i = pl.multiple_of(_vscal(offv, ii), 8)
            inx = jnp.minimum(i + 1, APW - 1)
            off_n = pl.multiple_of(_vscal(offv, inx), 8)

            @pl.when(i < APW)
            def _issue():
                pltpu.async_copy(w_hbm.at[pl.ds(off_i, 64), :], wb_c, sw_c)

                @pl.when(jnp.logical_not(win_ok))
                def _gath():
                    pltpu.async_copy(xl_hbm.at[cb_c.at[pl.ds(0, 64)]],
                                     xb_c, sx_c)

                @pl.when(i + 1 < APW)
                def _pref_col():
                    pltpu.async_copy(col_hbm.at[pl.ds(off_n, 64)],
                                     cb_p.at[pl.ds(0, 64)], sc_p)

            @pl.when(i >= 1)
            def _proc():
                j = i - 1
                cnt = _vscal(clv, j)
                en0 = jnp.minimum(cnt, 64)

                def edge_g(xbuf):
                    def edge(e, acc):
                        return tuple(
                            acc[f] + wb_p[e, pl.ds(f * 16, 16)]
                            * xbuf[e, pl.ds(f * 16, 16)]
                            for f in range(8))
                    return edge

                def edge_l(cbuf):
                    def edge(e, acc):
                        ce = cbuf[pl.ds(e, 16)][0] - jlo_eff
                        return tuple(
                            acc[f] + wb_p[e, pl.ds(f * 16, 16)]
                            * xwin[ce, pl.ds(f * 16, 16)]
                            for f in range(8))
                    return edge

                acc0 = tuple(jnp.zeros((16,), jnp.float32) for _ in range(8))
                acc = lax.cond(
                    win_ok,
                    lambda: lax.fori_loop(0, en0, edge_l(cb_p), acc0),
                    lambda: lax.fori_loop(0, en0, edge_g(xb_p), acc0))

                ngr = (cnt + 63) // 64

                def slow(g, acc):
                    off_j = pl.multiple_of(_vscal(offv, j), 8)
                    base = pl.multiple_of(off_j + g * 64, 8)
                    pltpu.sync_copy(w_hbm.at[pl.ds(base, 64), :], wb_p)
                    pltpu.sync_copy(col_hbm.at[pl.ds(base, 64)],
                                    cslow.at[pl.ds(0, 64)])

                    @pl.when(jnp.logical_not(win_ok))
                    def _g2():
                        pltpu.async_copy(xl_hbm.at[cslow.at[pl.ds(0, 64)]],
                                         xb_p, sslow).wait()
                    en = jnp.minimum(cnt - g * 64, 64)
                    return lax.cond(
                        win_ok,
                        lambda: lax.fori_loop(0, en, edge_l(cslow), acc),
                        lambda: lax.fori_loop(0, en, edge_g(xb_p), acc))
                acc = lax.fori_loop(1, ngr, slow, acc)
                for f in range(8):
                    aggv[j, pl.ds(f * 16, 16)] = acc[f]

            @pl.when(i < APW)
            def _finwait():
                pltpu.make_async_copy(
                    w_hbm.at[pl.ds(off_i, 64), :], wb_c, sw_c).wait()

                @pl.when(jnp.logical_not(win_ok))
                def _gw():
                    pltpu.make_async_copy(
                        xl_hbm.at[cb_c.at[pl.ds(0, 64)]], xb_c, sx_c).wait()

                @pl.when(i + 1 < APW)
                def _wait_col():
                    pltpu.make_async_copy(
                        col_hbm.at[pl.ds(off_n, 64)], cb_p.at[pl.ds(0, 64)],
                        sc_p).wait()

        def body(i, carry):
            lax.cond(
                i % 2 == 0,
                lambda: step(i, cb0, wb0, xb0, sw0, sx0,
                             cb1, wb1, xb1, sw1, sx1, sc1),
                lambda: step(i, cb1, wb1, xb1, sw1, sx1,
                             cb0, wb0, xb0, sw0, sx0, sc0))
            return carry
        lax.fori_loop(0, APW + 1, body, jnp.int32(0))
        pltpu.sync_copy(aggv, agg_hbm.at[pl.ds(a0, APW), :])
    return k


def _post_body(h_ref, agg_ref, l2w_ref, l2b_ref, ow_ref, ob_ref,
               n1w_ref, n1b_ref, h_out, xl_out):
    hc = jnp.dot(agg_ref[...], l2w_ref[...],
                 preferred_element_type=jnp.float32) + l2b_ref[...]
    hc = _ssp(hc)
    hc = jnp.dot(hc, ow_ref[...],
                 preferred_element_type=jnp.float32) + ob_ref[...]
    hn = h_ref[...] + hc
    h_out[...] = hn
    xl_out[...] = jnp.dot(hn, n1w_ref[...],
                          preferred_element_type=jnp.float32) + n1b_ref[...]


def _post_call(h, agg, l2w, l2b, ow, ob, n1w, n1b):
    return pl.pallas_call(
        _post_body,
        out_shape=(_SDS((N_ATOMS, HIDDEN_DIM), jnp.float32),
                   _SDS((N_ATOMS, NUM_FILTERS), jnp.float32)),
    )(h, agg, l2w, l2b, ow, ob, n1w, n1b)


def _final_body(h_ref, agg_ref, l2w_ref, l2b_ref, ow_ref, ob_ref,
                batch_ref, w0_ref, b0_ref, w1_ref, b1_ref, w2_ref, b2_ref,
                out_ref):
    hc = jnp.dot(agg_ref[...], l2w_ref[...],
                 preferred_element_type=jnp.float32) + l2b_ref[...]
    hc = _ssp(hc)
    hc = jnp.dot(hc, ow_ref[...],
                 preferred_element_type=jnp.float32) + ob_ref[...]
    hn = h_ref[...] + hc
    batch = batch_ref[...]
    gids = lax.broadcasted_iota(jnp.int32, (NUM_GRAPHS, N_ATOMS), 0)
    onehot = (batch == gids).astype(jnp.float32)
    hg = jnp.dot(onehot, hn, preferred_element_type=jnp.float32)
    o = _ssp(jnp.dot(hg, w0_ref[...],
                     preferred_element_type=jnp.float32) + b0_ref[...])
    o = _ssp(jnp.dot(o, w1_ref[...],
                     preferred_element_type=jnp.float32) + b1_ref[...])
    o = jax.nn.sigmoid(jnp.dot(o, w2_ref[...],
                               preferred_element_type=jnp.float32) + b2_ref[...])
    out_ref[...] = o


def _final_call(h, agg, l2w, l2b, ow, ob, batch, p):
    return pl.pallas_call(
        _final_body,
        out_shape=_SDS((NUM_GRAPHS, OUTPUT_DIM), jnp.float32),
    )(h, agg, l2w, l2b, ow, ob, batch.reshape(1, N_ATOMS),
      p['mlp0'][0], p['mlp0'][1].reshape(1, -1),
      p['mlp1'][0], p['mlp1'][1].reshape(1, -1),
      p['mlp2'][0], p['mlp2'][1].reshape(1, -1))


def kernel(x, pos, batch, params):
    bi = batch.astype(jnp.int32)
    ss = jnp.searchsorted(bi, bi, side='left').astype(jnp.int32)
    se = jnp.searchsorted(bi, bi, side='right').astype(jnp.int32)
    px = pos[:, 0] + 0.0
    py = pos[:, 1] + 0.0
    pz = pos[:, 2] + 0.0

    counts = _sc_count_fn()(px, py, pz, ss, se)

    p0 = params['inter'][0]
    h, xlin, offc, clc, nblk = _emb_call(
        x, params['emb'][0], params['emb'][1].reshape(1, -1),
        p0['lin1'][0], p0['lin1'][1].reshape(1, -1),
        counts.reshape(32, 128))
    offc = offc.reshape(-1)
    clc = clc.reshape(-1)
    nblk = nblk.reshape(1)

    col, d2 = _sc_build_fn()(px, py, pz, ss, se, offc, clc)
    d2r = d2.reshape(E_BUF // 128, 128)

    sc_agg = _sc_agg_fn()
    for l in range(NUM_INTER):
        p = params['inter'][l]
        W = _w_call(nblk, d2r,
                    p['f1'][0].T, p['f1'][1].reshape(-1, 1),
                    p['f2'][0].T, p['f2'][1].reshape(-1, 1))
        agg = sc_agg(xlin, W, col, offc, clc, ss, se)
        if l < NUM_INTER - 1:
            pn = params['inter'][l + 1]
            h, xlin = _post_call(
                h, agg, p['lin2'][0], p['lin2'][1].reshape(1, -1),
                p['out'][0], p['out'][1].reshape(1, -1),
                pn['lin1'][0], pn['lin1'][1].reshape(1, -1))
        else:
            out = _final_call(
                h, agg, p['lin2'][0], p['lin2'][1].reshape(1, -1),
                p['out'][0], p['out'][1].reshape(1, -1), bi, params)
    return out
